# baseline (speedup 1.0000x reference)
import functools

import jax
import jax.numpy as jnp
from jax import lax
from jax.experimental import pallas as pl
from jax.experimental.pallas import tpu as pltpu
from jax.experimental.pallas import tpu_sc as plsc

NC, NS = 2, 16
NW = NC * NS
B = 16384
D = 64
W2 = 2 * D
CHUNK = 128
ROWS_PER_W = B // NW
NCHUNK = ROWS_PER_W // CHUNK
IDX_ROWS = B // CHUNK

BLK = 2048


def _lrelu(x):
    return jnp.where(x >= 0, x, 0.3 * x)


@functools.cache
def _make_sc_gather():
    mesh = plsc.VectorSubcoreMesh(
        core_axis_name="c", subcore_axis_name="s", num_cores=NC, num_subcores=NS
    )

    @functools.partial(
        pl.kernel,
        mesh=mesh,
        out_type=(
            jax.ShapeDtypeStruct((B, W2), jnp.float32),
            jax.ShapeDtypeStruct((B, W2), jnp.float32),
            jax.ShapeDtypeStruct((B, W2), jnp.float32),
        ),
        scratch_types=(
            pltpu.VMEM((NCHUNK, CHUNK), jnp.int32),
            pltpu.VMEM((CHUNK, W2), jnp.float32),
            pltpu.SemaphoreType.DMA,
        ),
        compiler_params=pltpu.CompilerParams(use_tc_tiling_on_sc=True),
    )
    def _sc_gather(user_tab, item_tab, uidx, pidx, nidx, u_out, p_out, n_out,
                   idx_v, rows_v, sem):
        wid = lax.axis_index("s") * NC + lax.axis_index("c")
        row0 = wid * NCHUNK
        for tab, idx, out in ((user_tab, uidx, u_out),
                              (item_tab, pidx, p_out),
                              (item_tab, nidx, n_out)):
            pltpu.sync_copy(idx.at[pl.ds(row0, NCHUNK)], idx_v)
            for c in range(NCHUNK):
                pltpu.async_copy(tab.at[idx_v.at[c]], rows_v, sem).wait()
                pltpu.sync_copy(rows_v, out.at[pl.ds((row0 + c) * CHUNK, CHUNK)])

    return _sc_gather


def _tc_body(u_ref, p_ref, n_ref, up_ref, pp_ref, np_ref,
             w0_ref, b0_ref, w1_ref, b1_ref, w2_ref, b2_ref, o_ref):
    def pick(x_ref, par_ref):
        x = x_ref[...]
        par = par_ref[...]
        return jnp.where(par == 1, x[:, D:], x[:, :D])

    u = pick(u_ref, up_ref)
    p = pick(p_ref, pp_ref)
    n = pick(n_ref, np_ref)
    acc = jnp.sum(u * (p - n), axis=1, keepdims=True)
    for w_ref, b_ref in ((w0_ref, b0_ref), (w1_ref, b1_ref), (w2_ref, b2_ref)):
        w = w_ref[...]
        b = b_ref[...]
        u = _lrelu(jnp.dot(u, w, preferred_element_type=jnp.float32) + b)
        p = _lrelu(jnp.dot(p, w, preferred_element_type=jnp.float32) + b)
        n = _lrelu(jnp.dot(n, w, preferred_element_type=jnp.float32) + b)
        acc = acc + jnp.sum(u * (p - n), axis=1, keepdims=True)
    o_ref[...] = acc


_tc_score = pl.pallas_call(
    _tc_body,
    grid=(B // BLK,),
    in_specs=[
        pl.BlockSpec((BLK, W2), lambda i: (i, 0)),
        pl.BlockSpec((BLK, W2), lambda i: (i, 0)),
        pl.BlockSpec((BLK, W2), lambda i: (i, 0)),
        pl.BlockSpec((BLK, 1), lambda i: (i, 0)),
        pl.BlockSpec((BLK, 1), lambda i: (i, 0)),
        pl.BlockSpec((BLK, 1), lambda i: (i, 0)),
        pl.BlockSpec((D, D), lambda i: (0, 0)),
        pl.BlockSpec((1, D), lambda i: (0, 0)),
        pl.BlockSpec((D, D), lambda i: (0, 0)),
        pl.BlockSpec((1, D), lambda i: (0, 0)),
        pl.BlockSpec((D, D), lambda i: (0, 0)),
        pl.BlockSpec((1, D), lambda i: (0, 0)),
    ],
    out_specs=pl.BlockSpec((BLK, 1), lambda i: (i, 0)),
    out_shape=jax.ShapeDtypeStruct((B, 1), jnp.float32),
)


def kernel(user_indices, pos_item_indices, neg_item_indices, user_table,
           item_table, W1_0, b1_0, W1_1, b1_1, W1_2, b1_2):
    ut2 = user_table.reshape(-1, W2)
    it2 = item_table.reshape(-1, W2)
    ui = user_indices.astype(jnp.int32)
    pi = pos_item_indices.astype(jnp.int32)
    ni = neg_item_indices.astype(jnp.int32)
    uh = (ui // 2).reshape(IDX_ROWS, CHUNK)
    ph = (pi // 2).reshape(IDX_ROWS, CHUNK)
    nh = (ni // 2).reshape(IDX_ROWS, CHUNK)
    up = (ui % 2).reshape(B, 1)
    pp = (pi % 2).reshape(B, 1)
    np_ = (ni % 2).reshape(B, 1)
    u2, p2, n2 = _make_sc_gather()(ut2, it2, uh, ph, nh)
    return _tc_score(u2, p2, n2, up, pp, np_,
                     W1_0, b1_0.reshape(1, D),
                     W1_1, b1_1.reshape(1, D),
                     W1_2, b1_2.reshape(1, D))


# speedup vs baseline: 1.1067x; 1.1067x over previous
import functools

import jax
import jax.numpy as jnp
from jax import lax
from jax.experimental import pallas as pl
from jax.experimental.pallas import tpu as pltpu
from jax.experimental.pallas import tpu_sc as plsc

NC, NS = 2, 16
NW = NC * NS
B = 16384
D = 64
CHUNK = 128
ROWS_PER_W = B // NW
NCHUNK = ROWS_PER_W // CHUNK
IDX_ROWS = B // CHUNK

BLK = 4096


def _lrelu(x):
    return jnp.where(x >= 0, x, 0.3 * x)


@functools.cache
def _make_sc_gather():
    mesh = plsc.VectorSubcoreMesh(
        core_axis_name="c", subcore_axis_name="s", num_cores=NC, num_subcores=NS
    )

    @functools.partial(
        pl.kernel,
        mesh=mesh,
        out_type=(
            jax.ShapeDtypeStruct((B, D), jnp.float32),
            jax.ShapeDtypeStruct((B, D), jnp.float32),
            jax.ShapeDtypeStruct((B, D), jnp.float32),
        ),
        scratch_types=(
            pltpu.VMEM((3 * NCHUNK, CHUNK), jnp.int32),
            pltpu.VMEM((CHUNK, D), jnp.float32),
            pltpu.VMEM((CHUNK, D), jnp.float32),
            pltpu.SemaphoreType.DMA,
            pltpu.SemaphoreType.DMA,
        ),
        compiler_params=pltpu.CompilerParams(use_tc_tiling_on_sc=False),
    )
    def _sc_gather(user_tab, item_tab, uidx, pidx, nidx, u_out, p_out, n_out,
                   idx_v, rows0, rows1, sem0, sem1):
        wid = lax.axis_index("s") * NC + lax.axis_index("c")
        row0 = wid * NCHUNK
        bufs = ((rows0, sem0), (rows1, sem1))
        tabs = (user_tab, item_tab, item_tab)
        outs = (u_out, p_out, n_out)
        for s, idx in enumerate((uidx, pidx, nidx)):
            pltpu.sync_copy(idx.at[pl.ds(row0, NCHUNK)],
                            idx_v.at[pl.ds(s * NCHUNK, NCHUNK)])
        K = 3 * NCHUNK
        copies = [None] * K

        def writeback(k):
            copies[k].wait()
            s, c = divmod(k, NCHUNK)
            pltpu.sync_copy(bufs[k % 2][0],
                            outs[s].at[pl.ds((row0 + c) * CHUNK, CHUNK)])

        for k in range(K):
            s, _ = divmod(k, NCHUNK)
            buf, sem = bufs[k % 2]
            if k >= 2:
                writeback(k - 2)
            copies[k] = pltpu.async_copy(tabs[s].at[idx_v.at[k]], buf, sem)
        writeback(K - 2)
        writeback(K - 1)

    return _sc_gather


def _tc_body(u_ref, p_ref, n_ref, w0_ref, b0_ref, w1_ref, b1_ref,
             w2_ref, b2_ref, o_ref):
    u = u_ref[...]
    p = p_ref[...]
    n = n_ref[...]
    acc = jnp.sum(u * (p - n), axis=1, keepdims=True)
    for w_ref, b_ref in ((w0_ref, b0_ref), (w1_ref, b1_ref), (w2_ref, b2_ref)):
        w = w_ref[...].astype(jnp.bfloat16)
        b = b_ref[...]
        u = _lrelu(jnp.dot(u.astype(jnp.bfloat16), w,
                           preferred_element_type=jnp.float32) + b)
        p = _lrelu(jnp.dot(p.astype(jnp.bfloat16), w,
                           preferred_element_type=jnp.float32) + b)
        n = _lrelu(jnp.dot(n.astype(jnp.bfloat16), w,
                           preferred_element_type=jnp.float32) + b)
        acc = acc + jnp.sum(u * (p - n), axis=1, keepdims=True)
    o_ref[...] = acc


_tc_score = pl.pallas_call(
    _tc_body,
    grid=(B // BLK,),
    in_specs=[
        pl.BlockSpec((BLK, D), lambda i: (i, 0)),
        pl.BlockSpec((BLK, D), lambda i: (i, 0)),
        pl.BlockSpec((BLK, D), lambda i: (i, 0)),
        pl.BlockSpec((D, D), lambda i: (0, 0)),
        pl.BlockSpec((1, D), lambda i: (0, 0)),
        pl.BlockSpec((D, D), lambda i: (0, 0)),
        pl.BlockSpec((1, D), lambda i: (0, 0)),
        pl.BlockSpec((D, D), lambda i: (0, 0)),
        pl.BlockSpec((1, D), lambda i: (0, 0)),
    ],
    out_specs=pl.BlockSpec((BLK, 1), lambda i: (i, 0)),
    out_shape=jax.ShapeDtypeStruct((B, 1), jnp.float32),
)


def kernel(user_indices, pos_item_indices, neg_item_indices, user_table,
           item_table, W1_0, b1_0, W1_1, b1_1, W1_2, b1_2):
    ui = user_indices.astype(jnp.int32).reshape(IDX_ROWS, CHUNK)
    pi = pos_item_indices.astype(jnp.int32).reshape(IDX_ROWS, CHUNK)
    ni = neg_item_indices.astype(jnp.int32).reshape(IDX_ROWS, CHUNK)
    u, p, n = _make_sc_gather()(user_table, item_table, ui, pi, ni)
    return _tc_score(u, p, n,
                     W1_0, b1_0.reshape(1, D),
                     W1_1, b1_1.reshape(1, D),
                     W1_2, b1_2.reshape(1, D))


# speedup vs baseline: 1.5459x; 1.3968x over previous
import functools

import jax
import jax.numpy as jnp
from jax import lax
from jax.experimental import pallas as pl
from jax.experimental.pallas import tpu as pltpu
from jax.experimental.pallas import tpu_sc as plsc

NC, NS = 2, 16
NW = NC * NS
B = 16384
D = 64
CHUNK = 128
ROWS_PER_W = B // NW
NCHUNK = ROWS_PER_W // CHUNK
IDX_ROWS = B // CHUNK

BLK = 4096


def _lrelu(x):
    return jnp.where(x >= 0, x, 0.3 * x)


@functools.cache
def _make_sc_gather():
    mesh = plsc.VectorSubcoreMesh(
        core_axis_name="c", subcore_axis_name="s", num_cores=NC, num_subcores=NS
    )

    @functools.partial(
        pl.kernel,
        mesh=mesh,
        out_type=(
            jax.ShapeDtypeStruct((B, D), jnp.float32),
            jax.ShapeDtypeStruct((B, D), jnp.float32),
            jax.ShapeDtypeStruct((B, D), jnp.float32),
        ),
        scratch_types=(
            pltpu.VMEM((NCHUNK, CHUNK), jnp.int32),
            pltpu.VMEM((ROWS_PER_W, D), jnp.float32),
            pltpu.SemaphoreType.DMA,
        ),
        compiler_params=pltpu.CompilerParams(use_tc_tiling_on_sc=True),
    )
    def _sc_gather(user_tab, item_tab, uidx, pidx, nidx, u_out, p_out, n_out,
                   idx_v, rows_v, sem):
        wid = lax.axis_index("s") * NC + lax.axis_index("c")
        row0 = wid * NCHUNK
        tabs = (user_tab, item_tab, item_tab)
        outs = (u_out, p_out, n_out)
        for s, idx in enumerate((uidx, pidx, nidx)):
            tab = tabs[s]
            pltpu.sync_copy(idx.at[pl.ds(row0, NCHUNK)], idx_v)

            def issue(g, _):
                c = g // (CHUNK // 16)
                off = (g - c * (CHUNK // 16)) * 16
                vec = idx_v[c, pl.ds(off, 16)]
                for k in range(16):
                    i = vec[k]
                    pltpu.async_copy(tab.at[pl.ds(i, 1)],
                                     rows_v.at[pl.ds(g * 16 + k, 1)], sem)
                return _

            lax.fori_loop(0, ROWS_PER_W // 16, issue, 0)
            pltpu.make_async_copy(tab.at[pl.ds(0, ROWS_PER_W)], rows_v,
                                  sem).wait()
            pltpu.sync_copy(rows_v, outs[s].at[pl.ds(row0 * CHUNK, ROWS_PER_W)])

    return _sc_gather


def _tc_body(u_ref, p_ref, n_ref, w0_ref, b0_ref, w1_ref, b1_ref,
             w2_ref, b2_ref, o_ref):
    u = u_ref[...]
    p = p_ref[...]
    n = n_ref[...]
    acc = jnp.sum(u * (p - n), axis=1, keepdims=True)
    for w_ref, b_ref in ((w0_ref, b0_ref), (w1_ref, b1_ref), (w2_ref, b2_ref)):
        w = w_ref[...].astype(jnp.bfloat16)
        b = b_ref[...]
        u = _lrelu(jnp.dot(u.astype(jnp.bfloat16), w,
                           preferred_element_type=jnp.float32) + b)
        p = _lrelu(jnp.dot(p.astype(jnp.bfloat16), w,
                           preferred_element_type=jnp.float32) + b)
        n = _lrelu(jnp.dot(n.astype(jnp.bfloat16), w,
                           preferred_element_type=jnp.float32) + b)
        acc = acc + jnp.sum(u * (p - n), axis=1, keepdims=True)
    o_ref[...] = acc


_tc_score = pl.pallas_call(
    _tc_body,
    grid=(B // BLK,),
    in_specs=[
        pl.BlockSpec((BLK, D), lambda i: (i, 0)),
        pl.BlockSpec((BLK, D), lambda i: (i, 0)),
        pl.BlockSpec((BLK, D), lambda i: (i, 0)),
        pl.BlockSpec((D, D), lambda i: (0, 0)),
        pl.BlockSpec((1, D), lambda i: (0, 0)),
        pl.BlockSpec((D, D), lambda i: (0, 0)),
        pl.BlockSpec((1, D), lambda i: (0, 0)),
        pl.BlockSpec((D, D), lambda i: (0, 0)),
        pl.BlockSpec((1, D), lambda i: (0, 0)),
    ],
    out_specs=pl.BlockSpec((BLK, 1), lambda i: (i, 0)),
    out_shape=jax.ShapeDtypeStruct((B, 1), jnp.float32),
)


def kernel(user_indices, pos_item_indices, neg_item_indices, user_table,
           item_table, W1_0, b1_0, W1_1, b1_1, W1_2, b1_2):
    ui = user_indices.astype(jnp.int32).reshape(IDX_ROWS, CHUNK)
    pi = pos_item_indices.astype(jnp.int32).reshape(IDX_ROWS, CHUNK)
    ni = neg_item_indices.astype(jnp.int32).reshape(IDX_ROWS, CHUNK)
    u, p, n = _make_sc_gather()(user_table, item_table, ui, pi, ni)
    return _tc_score(u, p, n,
                     W1_0, b1_0.reshape(1, D),
                     W1_1, b1_1.reshape(1, D),
                     W1_2, b1_2.reshape(1, D))


# speedup vs baseline: 1.6384x; 1.0598x over previous
import functools

import jax
import jax.numpy as jnp
from jax import lax
from jax.experimental import pallas as pl
from jax.experimental.pallas import tpu as pltpu
from jax.experimental.pallas import tpu_sc as plsc

NC, NS = 2, 16
NW = NC * NS
B = 16384
D = 64
CHUNK = 128
IDX_ROWS = B // CHUNK

BLK = 4096


@functools.cache
def _make_sc_gather(nstream):
    mesh = plsc.VectorSubcoreMesh(
        core_axis_name="c", subcore_axis_name="s", num_cores=NC, num_subcores=NS
    )
    rows_per_w = B // NW
    nchunk = rows_per_w // CHUNK

    @functools.partial(
        pl.kernel,
        mesh=mesh,
        out_type=tuple(
            jax.ShapeDtypeStruct((B, D), jnp.float32) for _ in range(nstream)
        ),
        scratch_types=(
            pltpu.VMEM((nchunk, CHUNK), jnp.int32),
            pltpu.VMEM((rows_per_w, D), jnp.float32),
            pltpu.SemaphoreType.DMA,
        ),
        compiler_params=pltpu.CompilerParams(use_tc_tiling_on_sc=True),
    )
    def _sc_gather(tab, *args):
        idxs = args[:nstream]
        outs = args[nstream:2 * nstream]
        idx_v, rows_v, sem = args[2 * nstream:]
        wid = lax.axis_index("s") * NC + lax.axis_index("c")
        row0 = wid * nchunk
        for s in range(nstream):
            pltpu.sync_copy(idxs[s].at[pl.ds(row0, nchunk)], idx_v)

            def issue(g, _):
                c = g // (CHUNK // 16)
                off = (g - c * (CHUNK // 16)) * 16
                vec = idx_v[c, pl.ds(off, 16)]
                for k in range(16):
                    i = vec[k]
                    pltpu.async_copy(tab.at[pl.ds(i, 1)],
                                     rows_v.at[pl.ds(g * 16 + k, 1)], sem)
                return _

            lax.fori_loop(0, rows_per_w // 16, issue, 0)
            pltpu.make_async_copy(tab.at[pl.ds(0, rows_per_w)], rows_v,
                                  sem).wait()
            pltpu.sync_copy(rows_v, outs[s].at[pl.ds(row0 * CHUNK, rows_per_w)])

    return _sc_gather


def _tc_body(u_ref, p_ref, n_ref, w0_ref, b0_ref, w1_ref, b1_ref,
             w2_ref, b2_ref, o_ref):
    u = u_ref[...]
    p = p_ref[...]
    n = n_ref[...]
    acc = jnp.sum(u * (p - n), axis=1, keepdims=True)
    x = jnp.concatenate([u, p, n], axis=0).astype(jnp.bfloat16)
    for w_ref, b_ref in ((w0_ref, b0_ref), (w1_ref, b1_ref), (w2_ref, b2_ref)):
        w = w_ref[...].astype(jnp.bfloat16)
        b = b_ref[...]
        y = jnp.dot(x, w, preferred_element_type=jnp.float32) + b
        yf = jnp.maximum(y, 0.3 * y)
        x = yf.astype(jnp.bfloat16)
        uf = yf[:BLK]
        pf = yf[BLK:2 * BLK]
        nf = yf[2 * BLK:]
        acc = acc + jnp.sum(uf * (pf - nf), axis=1, keepdims=True)
    o_ref[...] = acc


_tc_score = pl.pallas_call(
    _tc_body,
    grid=(B // BLK,),
    in_specs=[
        pl.BlockSpec((BLK, D), lambda i: (i, 0)),
        pl.BlockSpec((BLK, D), lambda i: (i, 0)),
        pl.BlockSpec((BLK, D), lambda i: (i, 0)),
        pl.BlockSpec((D, D), lambda i: (0, 0)),
        pl.BlockSpec((1, D), lambda i: (0, 0)),
        pl.BlockSpec((D, D), lambda i: (0, 0)),
        pl.BlockSpec((1, D), lambda i: (0, 0)),
        pl.BlockSpec((D, D), lambda i: (0, 0)),
        pl.BlockSpec((1, D), lambda i: (0, 0)),
    ],
    out_specs=pl.BlockSpec((BLK, 1), lambda i: (i, 0)),
    out_shape=jax.ShapeDtypeStruct((B, 1), jnp.float32),
)


def kernel(user_indices, pos_item_indices, neg_item_indices, user_table,
           item_table, W1_0, b1_0, W1_1, b1_1, W1_2, b1_2):
    ui = user_indices.astype(jnp.int32).reshape(IDX_ROWS, CHUNK)
    pi = pos_item_indices.astype(jnp.int32).reshape(IDX_ROWS, CHUNK)
    ni = neg_item_indices.astype(jnp.int32).reshape(IDX_ROWS, CHUNK)
    (u,) = _make_sc_gather(1)(user_table, ui)
    p, n = _make_sc_gather(2)(item_table, pi, ni)
    return _tc_score(u, p, n,
                     W1_0, b1_0.reshape(1, D),
                     W1_1, b1_1.reshape(1, D),
                     W1_2, b1_2.reshape(1, D))
